# Initial kernel scaffold; baseline (speedup 1.0000x reference)
import functools

import jax
import jax.numpy as jnp
from jax import lax
from jax.experimental import pallas as pl
from jax.experimental.pallas import tpu as pltpu
from jax.experimental.pallas import tpu_sc as plsc

CHANNELS = 128
HALF = CHANNELS // 2
MAX_LEN = 1000
BATCH = 16384
LANES = 16
CHUNK = 128


def kernel(spatial_coord, pos_enc):
    tbl = pos_enc.reshape(2 * MAX_LEN, HALF)
    coords = spatial_coord.reshape(2 * BATCH)

    info = plsc.get_sparse_core_info()
    nw = info.num_cores * info.num_subcores
    n2 = (2 * BATCH) // nw
    nchunk = n2 // CHUNK

    @functools.partial(
        pl.kernel,
        mesh=plsc.VectorSubcoreMesh(core_axis_name="c", subcore_axis_name="s"),
        out_type=jax.ShapeDtypeStruct((2 * BATCH, HALF), jnp.float32),
        scratch_types=[
            pltpu.VMEM((n2,), jnp.int32),
            pltpu.VMEM((nchunk, CHUNK), jnp.int32),
            pltpu.VMEM((n2, HALF), jnp.float32),
            pltpu.SemaphoreType.DMA,
        ],
    )
    def run(coords_hbm, tbl_hbm, out_hbm, coords_v, idx_v, rows_v, sem):
        wid = lax.axis_index("s") * info.num_cores + lax.axis_index("c")
        base = wid * n2
        pltpu.sync_copy(coords_hbm.at[pl.ds(base, n2)], coords_v)

        parity = lax.iota(jnp.int32, LANES) & 1
        per_chunk = CHUNK // LANES
        for i in range(n2 // LANES):
            v = coords_v[pl.ds(i * LANES, LANES)]
            idx_v[i // per_chunk, pl.ds((i % per_chunk) * LANES, LANES)] = (
                v * 2 + parity
            )

        copies = [
            pltpu.async_copy(
                tbl_hbm.at[idx_v.at[j]],
                rows_v.at[pl.ds(j * CHUNK, CHUNK)],
                sem,
            )
            for j in range(nchunk)
        ]
        for c in copies:
            c.wait()
        pltpu.sync_copy(rows_v, out_hbm.at[pl.ds(base, n2)])

    return run(coords, tbl).reshape(BATCH, 1, CHANNELS)


# speedup vs baseline: 2363.5565x; 2363.5565x over previous
import functools

import jax
import jax.numpy as jnp
from jax import lax
from jax.experimental import pallas as pl
from jax.experimental.pallas import tpu as pltpu
from jax.experimental.pallas import tpu_sc as plsc

CHANNELS = 128
HALF = CHANNELS // 2
MAX_LEN = 1000
BATCH = 16384
LANES = 16
CHUNK = 128


def kernel(spatial_coord, pos_enc):
    tbl = pos_enc.reshape(2 * MAX_LEN, HALF)
    coords = spatial_coord.reshape(2 * BATCH)

    info = plsc.get_sparse_core_info()
    nw = info.num_cores * info.num_subcores
    n2 = (2 * BATCH) // nw
    nchunk = n2 // CHUNK

    @functools.partial(
        pl.kernel,
        mesh=plsc.VectorSubcoreMesh(core_axis_name="c", subcore_axis_name="s"),
        out_type=jax.ShapeDtypeStruct((2 * BATCH, HALF), jnp.float32),
        scratch_types=[
            pltpu.VMEM((n2,), jnp.int32),
            pltpu.VMEM((nchunk, CHUNK), jnp.int32),
            pltpu.VMEM((n2, HALF), jnp.float32),
            pltpu.SemaphoreType.DMA,
        ],
        compiler_params=pltpu.CompilerParams(use_tc_tiling_on_sc=False),
    )
    def run(coords_hbm, tbl_hbm, out_hbm, coords_v, idx_v, rows_v, sem):
        wid = lax.axis_index("s") * info.num_cores + lax.axis_index("c")
        base = wid * n2
        pltpu.sync_copy(coords_hbm.at[pl.ds(base, n2)], coords_v)

        parity = lax.iota(jnp.int32, LANES) & 1
        per_chunk = CHUNK // LANES
        for i in range(n2 // LANES):
            v = coords_v[pl.ds(i * LANES, LANES)]
            idx_v[i // per_chunk, pl.ds((i % per_chunk) * LANES, LANES)] = (
                v * 2 + parity
            )

        copies = [
            pltpu.async_copy(
                tbl_hbm.at[idx_v.at[j]],
                rows_v.at[pl.ds(j * CHUNK, CHUNK)],
                sem,
            )
            for j in range(nchunk)
        ]
        for c in copies:
            c.wait()
        pltpu.sync_copy(rows_v, out_hbm.at[pl.ds(base, n2)])

    return run(coords, tbl).reshape(BATCH, 1, CHANNELS)
